# baseline (speedup 1.0000x reference)
import functools
import jax
import jax.numpy as jnp
from jax import lax
from jax.experimental import pallas as pl
from jax.experimental.pallas import tpu as pltpu
from jax.experimental.pallas import tpu_sc as plsc

N = 10000
E = 320000
NG = 64
NC = 2
NS = 16
NW = NC * NS
EPW = E // NW
CH = 128
NFULL = EPW // CH
TAIL = EPW - NFULL * CH
E4 = E // 4


def _node_transform(parts, W1, b1):
    P, n, F = parts.shape
    H = W1.shape[1]
    BN = 1000

    def body(p_ref, w_ref, b_ref, a_ref, bout_ref):
        h = jnp.max(p_ref[...], axis=0)
        w1a = w_ref[:F, :]
        w1b = w_ref[F:, :]
        a_ref[...] = h @ (w1a - w1b) + b_ref[...]
        bout_ref[...] = h @ w1b

    return pl.pallas_call(
        body,
        grid=(n // BN,),
        in_specs=[
            pl.BlockSpec((P, BN, F), lambda i: (0, i, 0)),
            pl.BlockSpec((2 * F, H), lambda i: (0, 0)),
            pl.BlockSpec((1, H), lambda i: (0, 0)),
        ],
        out_specs=[
            pl.BlockSpec((BN, H), lambda i: (i, 0)),
            pl.BlockSpec((BN, H), lambda i: (i, 0)),
        ],
        out_shape=[
            jax.ShapeDtypeStruct((n, H), jnp.float32),
            jax.ShapeDtypeStruct((n, H), jnp.float32),
        ],
    )(parts, W1, b1.reshape(1, H))


def _sc_gather_combine(A, B, src, dst):
    H = A.shape[1]
    mesh = plsc.VectorSubcoreMesh(core_axis_name="c", subcore_axis_name="s")

    NB = 4

    @functools.partial(
        pl.kernel,
        out_type=pltpu.HBM((E, H), jnp.float32),
        mesh=mesh,
        scratch_types=[
            pltpu.VMEM((EPW,), jnp.int32),
            pltpu.VMEM((EPW,), jnp.int32),
            [pltpu.VMEM((CH, H), jnp.float32) for _ in range(NB)],
            [pltpu.SemaphoreType.DMA for _ in range(NB)],
            [pltpu.SemaphoreType.DMA for _ in range(NB)],
            [pltpu.SemaphoreType.DMA for _ in range(NB)],
            pltpu.VMEM((TAIL, H), jnp.float32),
            pltpu.SemaphoreType.DMA,
        ],
        compiler_params=pltpu.CompilerParams(use_tc_tiling_on_sc=False),
    )
    def k(a_h, b_h, src_h, dst_h, out_h,
          ids, iss, ras, sAs, sBs, sWs, tra, sT):
        wid = lax.axis_index("s") * NC + lax.axis_index("c")
        ebase = wid * EPW
        pltpu.sync_copy(dst_h.at[pl.ds(ebase, EPW)], ids)
        pltpu.sync_copy(src_h.at[pl.ds(ebase, EPW)], iss)

        def fire_a(ci, p):
            pltpu.async_copy(a_h.at[ids.at[pl.ds(ci * CH, CH)]], ras[p], sAs[p])

        def wait_a_fire_b(ci, p):
            pltpu.make_async_copy(a_h.at[ids.at[pl.ds(ci * CH, CH)]], ras[p], sAs[p]).wait()
            pltpu.async_copy(b_h.at[iss.at[pl.ds(ci * CH, CH)]], ras[p], sBs[p], add=True)

        def wait_b_fire_w(ci, p):
            pltpu.make_async_copy(b_h.at[iss.at[pl.ds(ci * CH, CH)]], ras[p], sBs[p]).wait()
            pltpu.async_copy(ras[p], out_h.at[pl.ds(ebase + ci * CH, CH)], sWs[p])

        def wait_w(ci, p):
            pltpu.make_async_copy(ras[p], out_h.at[pl.ds(ebase + ci * CH, CH)], sWs[p]).wait()

        for p in range(NB):
            fire_a(p, p)

        NMACRO = NFULL // NB
        def macro(it, _):
            ci0 = it * NB
            for p in range(NB):
                wait_a_fire_b(ci0 + p, p)
            for p in range(NB):
                wait_b_fire_w(ci0 + p, p)
            for p in range(NB):
                ci = ci0 + p + NB

                @pl.when(ci < NFULL)
                def _(ci=ci, p=p):
                    wait_w(ci - NB, p)
                    fire_a(ci, p)

            return _

        lax.fori_loop(0, NMACRO, macro, None)
        for ci, p in ((NFULL - 2, 0), (NFULL - 1, 1)):
            wait_a_fire_b(ci, p)
        for ci, p in ((NFULL - 2, 0), (NFULL - 1, 1)):
            wait_b_fire_w(ci, p)
        for ci, p in ((NFULL - 4, 2), (NFULL - 3, 3), (NFULL - 2, 0), (NFULL - 1, 1)):
            wait_w(ci, p)

        base = ebase + NFULL * CH
        pltpu.async_copy(a_h.at[ids.at[pl.ds(NFULL * CH, TAIL)]], tra, sT)
        pltpu.make_async_copy(a_h.at[ids.at[pl.ds(NFULL * CH, TAIL)]], tra, sT).wait()
        pltpu.async_copy(b_h.at[iss.at[pl.ds(NFULL * CH, TAIL)]], tra, sT, add=True)
        pltpu.make_async_copy(b_h.at[iss.at[pl.ds(NFULL * CH, TAIL)]], tra, sT).wait()
        pltpu.sync_copy(tra, out_h.at[pl.ds(base, TAIL)])

    return k(A, B, src, dst)


def _edge_mlp_packed(m1p, W2big, b2big):
    PW = W2big.shape[1]
    BE4 = 2000

    def body(x_ref, w_ref, bb_ref, o_ref):
        m1 = jnp.maximum(x_ref[...], 0.0)
        o_ref[...] = jnp.maximum(m1 @ w_ref[...] + bb_ref[...], 0.0)

    return pl.pallas_call(
        body,
        grid=(E4 // BE4,),
        in_specs=[
            pl.BlockSpec((BE4, 128), lambda i: (i, 0)),
            pl.BlockSpec((128, PW), lambda i: (0, 0)),
            pl.BlockSpec((1, PW), lambda i: (0, 0)),
        ],
        out_specs=pl.BlockSpec((BE4, PW), lambda i: (i, 0)),
        out_shape=jax.ShapeDtypeStruct((E4, PW), jnp.float32),
    )(m1p, W2big, b2big)


def _sc_segmax(msg_p, dst, feature_split=False):
    PW = msg_p.shape[1]
    Fh = 16
    npass = 2
    RANGE = N // npass
    CHS = 400
    CHS4 = CHS // 4
    EPWk = (E // NS) if feature_split else EPW
    NCHS = EPWk // CHS
    MSUB = RANGE // 16
    MLAST = RANGE - 15 * MSUB
    WS = 2
    mesh = plsc.VectorSubcoreMesh(core_axis_name="c", subcore_axis_name="s")

    NBUF = 4

    @functools.partial(
        pl.kernel,
        out_type=pltpu.HBM((NC, N, Fh), jnp.float32),
        mesh=mesh,
        scratch_types=[
            [pltpu.VMEM((CHS,), jnp.int32) for _ in range(NBUF)],
            [pltpu.VMEM((CHS4, 4, Fh), jnp.float32) for _ in range(NBUF)],
            pltpu.VMEM((RANGE + 8, Fh), jnp.float32),
            pltpu.VMEM((MLAST, Fh), jnp.float32),
            pltpu.VMEM((MLAST, Fh), jnp.float32),
            pltpu.VMEM_SHARED((WS, RANGE, Fh), jnp.float32),
            [pltpu.SemaphoreType.DMA for _ in range(NBUF)],
            [pltpu.SemaphoreType.DMA for _ in range(NBUF)],
        ],
        compiler_params=pltpu.CompilerParams(use_tc_tiling_on_sc=False),
    )
    def k(msg_h, dst_h, parts_h, dvs, mvs, acc, sbuf, mbuf, shm, sDs, sMs):
        cid = lax.axis_index("c")
        sid = lax.axis_index("s")
        if feature_split:
            ebase = sid * EPWk
            colofs = cid * Fh
        else:
            ebase = (sid * NC + cid) * EPWk
            colofs = 0
        rbase = (ebase // 4)

        def fire(ci, p):
            pltpu.async_copy(dst_h.at[pl.ds(ebase + ci * CHS, CHS)], dvs[p], sDs[p])
            pltpu.async_copy(
                msg_h.at[pl.ds(rbase + ci * CHS4, CHS4), :, pl.ds(colofs, Fh)],
                mvs[p], sMs[p])

        def wait_in(ci, p):
            pltpu.make_async_copy(dst_h.at[pl.ds(ebase + ci * CHS, CHS)], dvs[p], sDs[p]).wait()
            pltpu.make_async_copy(
                msg_h.at[pl.ds(rbase + ci * CHS4, CHS4), :, pl.ds(colofs, Fh)],
                mvs[p], sMs[p]).wait()

        def do_pass(pp, _):
            nbase = pp * RANGE

            def zero(i, _):
                acc[i, pl.ds(0, 16)] = jnp.zeros((16,), jnp.float32)
                return _

            lax.fori_loop(0, RANGE + 8, zero, None)

            def rmw_chunk(ci, p):
                dv = dvs[p]
                mv = mvs[p]

                def group(gi, _):
                    dvec = dv[pl.ds(gi * 16, 16)] - nbase
                    inr = jnp.logical_and(dvec >= 0, dvec < RANGE)
                    lsel = jnp.where(inr, dvec, RANGE)
                    for lane in range(16):
                        local = lsel[lane]
                        r = gi * 4 + (lane // 4)
                        old = acc[local, pl.ds(0, 16)]
                        row = mv[r, lane % 4, pl.ds(0, 16)]
                        acc[local, pl.ds(0, 16)] = jnp.maximum(old, row)

                    return _

                lax.fori_loop(0, CHS // 16, group, None)

            for p in range(NBUF):
                fire(p, p)

            def macro(it, _):
                ci0 = it * NBUF
                for p in range(NBUF):
                    ci = ci0 + p
                    wait_in(ci, p)
                    rmw_chunk(ci, p)

                    @pl.when(ci + NBUF < NCHS)
                    def _(ci=ci, p=p):
                        fire(ci + NBUF, p)

                return _

            lax.fori_loop(0, NCHS // NBUF, macro, None)
            for r in range(NCHS % NBUF):
                ci = (NCHS // NBUF) * NBUF + r
                wait_in(ci, r)
                rmw_chunk(ci, r)

            roff = sid * MSUB

            def merge_rows(nrows, roff, zero_first):
                if zero_first:
                    def mz(i, _):
                        mbuf[i, pl.ds(0, 16)] = jnp.zeros((16,), jnp.float32)
                        return _

                    lax.fori_loop(0, nrows, mz, None)

                def slot(t, _):
                    pltpu.sync_copy(shm.at[t, pl.ds(roff, nrows)],
                                    sbuf.at[pl.ds(0, nrows)])

                    def row(i, _):
                        mbuf[i, pl.ds(0, 16)] = jnp.maximum(
                            mbuf[i, pl.ds(0, 16)],
                            sbuf[i, pl.ds(0, 16)])
                        return _

                    lax.fori_loop(0, nrows, row, None)
                    return _

                lax.fori_loop(0, WS, slot, None)

            for w in range(NS // WS):
                @pl.when(sid // WS == w)
                def _(w=w):
                    pltpu.sync_copy(acc.at[pl.ds(0, RANGE)], shm.at[sid % WS])

                plsc.subcore_barrier()

                @pl.when(sid < 15)
                def _(w=w):
                    merge_rows(MSUB, roff, w == 0)

                @pl.when(sid == 15)
                def _(w=w):
                    merge_rows(MLAST, roff, w == 0)

                plsc.subcore_barrier()

            @pl.when(sid < 15)
            def _():
                pltpu.sync_copy(mbuf.at[pl.ds(0, MSUB)],
                                parts_h.at[cid, pl.ds(nbase + roff, MSUB)])

            @pl.when(sid == 15)
            def _():
                pltpu.sync_copy(mbuf.at[pl.ds(0, MLAST)],
                                parts_h.at[cid, pl.ds(nbase + roff, MLAST)])
            return _

        lax.fori_loop(0, npass, do_pass, None)

    return k(msg_p.reshape(E4, 4, PW // 4), dst)


def _head(parts3, batch2d, lin1_W, lin1_b, lin2_W, lin2_b):
    P, n, Fhp = parts3.shape
    BN = 1000
    steps = n // BN
    Fh = P * Fhp
    F1 = lin1_W.shape[1]
    F2 = lin2_W.shape[1]

    def body(p_ref, b_ref, w1_ref, b1_ref, w2_ref, b2_ref, o_ref, sums, cnts):
        i = pl.program_id(0)

        @pl.when(i == 0)
        def _init():
            sums[...] = jnp.zeros_like(sums)
            cnts[...] = jnp.zeros_like(cnts)

        h = jnp.concatenate([p_ref[0], p_ref[1]], axis=-1)
        bb = b_ref[...]
        ids = lax.broadcasted_iota(jnp.int32, (1, NG), 1).astype(jnp.float32)
        oh = (bb == ids).astype(jnp.float32)
        dn = (((0,), (0,)), ((), ()))
        sums[...] += lax.dot_general(oh, h, dn)
        cnts[...] += lax.dot_general(oh, jnp.ones((BN, 1), jnp.float32), dn)

        @pl.when(i == steps - 1)
        def _fin():
            g = sums[...] / jnp.maximum(cnts[...], 1.0)
            z = jnp.maximum(g @ w1_ref[...] + b1_ref[...], 0.0)
            z = jnp.maximum(z @ w2_ref[...] + b2_ref[...], 0.0)
            m = jnp.max(z, axis=-1, keepdims=True)
            lse = m + jnp.log(jnp.sum(jnp.exp(z - m), axis=-1, keepdims=True))
            o_ref[...] = z - lse

    return pl.pallas_call(
        body,
        grid=(steps,),
        in_specs=[
            pl.BlockSpec((P, BN, Fhp), lambda i: (0, i, 0)),
            pl.BlockSpec((BN, 1), lambda i: (i, 0)),
            pl.BlockSpec((Fh, F1), lambda i: (0, 0)),
            pl.BlockSpec((1, F1), lambda i: (0, 0)),
            pl.BlockSpec((F1, F2), lambda i: (0, 0)),
            pl.BlockSpec((1, F2), lambda i: (0, 0)),
        ],
        out_specs=pl.BlockSpec((NG, F2), lambda i: (0, 0)),
        out_shape=jax.ShapeDtypeStruct((NG, F2), jnp.float32),
        scratch_shapes=[
            pltpu.VMEM((NG, Fh), jnp.float32),
            pltpu.VMEM((NG, 1), jnp.float32),
        ],
    )(parts3, batch2d, lin1_W, lin1_b.reshape(1, F1), lin2_W, lin2_b.reshape(1, F2))


def _big_w2(W2, b2):
    H, F2 = W2.shape
    Wb = jnp.zeros((128, 4 * F2), jnp.float32)
    for j in range(4):
        Wb = Wb.at[j * H:(j + 1) * H, j * F2:(j + 1) * F2].set(W2)
    bb = jnp.tile(b2, (4,)).reshape(1, 4 * F2)
    return Wb, bb


def _layer(parts, src, dst, W1, b1, W2, b2, feature_split):
    A, B = _node_transform(parts, W1, b1)
    m1raw = _sc_gather_combine(A, B, src, dst)
    m1p = m1raw.reshape(E4, 128)
    W2big, b2big = _big_w2(W2, b2)
    msg_p = _edge_mlp_packed(m1p, W2big, b2big)
    return _sc_segmax(msg_p, dst, feature_split)


@jax.jit
def kernel(x, edge_index, batch,
           eg1_W1, eg1_b1, eg1_W2, eg1_b2,
           eg2_W1, eg2_b1, eg2_W2, eg2_b2,
           eg3_W1, eg3_b1, eg3_W2, eg3_b2,
           lin1_W, lin1_b, lin2_W, lin2_b):
    src = edge_index[0]
    dst = edge_index[1]
    parts = x.reshape(1, N, x.shape[1])
    parts = _layer(parts, src, dst, eg1_W1, eg1_b1, eg1_W2, eg1_b2, False)
    parts = _layer(parts, src, dst, eg2_W1, eg2_b1, eg2_W2, eg2_b2, False)
    parts = _layer(parts, src, dst, eg3_W1, eg3_b1, eg3_W2, eg3_b2, True)
    batch2d = batch.astype(jnp.float32).reshape(N, 1)
    return _head(parts, batch2d, lin1_W, lin1_b, lin2_W, lin2_b)


# speedup vs baseline: 1.4046x; 1.4046x over previous
import functools
import jax
import jax.numpy as jnp
from jax import lax
from jax.experimental import pallas as pl
from jax.experimental.pallas import tpu as pltpu
from jax.experimental.pallas import tpu_sc as plsc

N = 10000
E = 320000
NG = 64
NC = 2
NS = 16
NW = NC * NS
EPW = E // NW
CH = 128
NFULL = EPW // CH
TAIL = EPW - NFULL * CH
E4 = E // 4


def _node_transform(parts, W1, b1):
    P, n, F = parts.shape
    H = W1.shape[1]
    BN = 1000

    def body(p_ref, w_ref, b_ref, a_ref, bout_ref):
        h = jnp.max(p_ref[...], axis=0)
        w1a = w_ref[:F, :]
        w1b = w_ref[F:, :]
        a_ref[...] = h @ (w1a - w1b) + b_ref[...]
        bout_ref[...] = h @ w1b

    return pl.pallas_call(
        body,
        grid=(n // BN,),
        in_specs=[
            pl.BlockSpec((P, BN, F), lambda i: (0, i, 0)),
            pl.BlockSpec((2 * F, H), lambda i: (0, 0)),
            pl.BlockSpec((1, H), lambda i: (0, 0)),
        ],
        out_specs=[
            pl.BlockSpec((BN, H), lambda i: (i, 0)),
            pl.BlockSpec((BN, H), lambda i: (i, 0)),
        ],
        out_shape=[
            jax.ShapeDtypeStruct((n, H), jnp.float32),
            jax.ShapeDtypeStruct((n, H), jnp.float32),
        ],
    )(parts, W1, b1.reshape(1, H))


def _sc_gather_combine(A, B, src, dst):
    H = A.shape[1]
    mesh = plsc.VectorSubcoreMesh(core_axis_name="c", subcore_axis_name="s")

    NB = 4

    @functools.partial(
        pl.kernel,
        out_type=pltpu.HBM((E, H), jnp.float32),
        mesh=mesh,
        scratch_types=[
            pltpu.VMEM((EPW,), jnp.int32),
            pltpu.VMEM((EPW,), jnp.int32),
            [pltpu.VMEM((CH, H), jnp.float32) for _ in range(NB)],
            [pltpu.SemaphoreType.DMA for _ in range(NB)],
            [pltpu.SemaphoreType.DMA for _ in range(NB)],
            [pltpu.SemaphoreType.DMA for _ in range(NB)],
            pltpu.VMEM((TAIL, H), jnp.float32),
            pltpu.SemaphoreType.DMA,
        ],
        compiler_params=pltpu.CompilerParams(use_tc_tiling_on_sc=False),
    )
    def k(a_h, b_h, src_h, dst_h, out_h,
          ids, iss, ras, sAs, sBs, sWs, tra, sT):
        wid = lax.axis_index("s") * NC + lax.axis_index("c")
        ebase = wid * EPW
        pltpu.sync_copy(dst_h.at[pl.ds(ebase, EPW)], ids)
        pltpu.sync_copy(src_h.at[pl.ds(ebase, EPW)], iss)

        def fire_a(ci, p):
            pltpu.async_copy(a_h.at[ids.at[pl.ds(ci * CH, CH)]], ras[p], sAs[p])

        def wait_a_fire_b(ci, p):
            pltpu.make_async_copy(a_h.at[ids.at[pl.ds(ci * CH, CH)]], ras[p], sAs[p]).wait()
            pltpu.async_copy(b_h.at[iss.at[pl.ds(ci * CH, CH)]], ras[p], sBs[p], add=True)

        def wait_b_fire_w(ci, p):
            pltpu.make_async_copy(b_h.at[iss.at[pl.ds(ci * CH, CH)]], ras[p], sBs[p]).wait()
            pltpu.async_copy(ras[p], out_h.at[pl.ds(ebase + ci * CH, CH)], sWs[p])

        def wait_w(ci, p):
            pltpu.make_async_copy(ras[p], out_h.at[pl.ds(ebase + ci * CH, CH)], sWs[p]).wait()

        for p in range(NB):
            fire_a(p, p)

        NMACRO = NFULL // NB
        def macro(it, _):
            ci0 = it * NB
            for p in range(NB):
                wait_a_fire_b(ci0 + p, p)
            for p in range(NB):
                wait_b_fire_w(ci0 + p, p)
            for p in range(NB):
                ci = ci0 + p + NB

                @pl.when(ci < NFULL)
                def _(ci=ci, p=p):
                    wait_w(ci - NB, p)
                    fire_a(ci, p)

            return _

        lax.fori_loop(0, NMACRO, macro, None)
        for ci, p in ((NFULL - 2, 0), (NFULL - 1, 1)):
            wait_a_fire_b(ci, p)
        for ci, p in ((NFULL - 2, 0), (NFULL - 1, 1)):
            wait_b_fire_w(ci, p)
        for ci, p in ((NFULL - 4, 2), (NFULL - 3, 3), (NFULL - 2, 0), (NFULL - 1, 1)):
            wait_w(ci, p)

        base = ebase + NFULL * CH
        pltpu.async_copy(a_h.at[ids.at[pl.ds(NFULL * CH, TAIL)]], tra, sT)
        pltpu.make_async_copy(a_h.at[ids.at[pl.ds(NFULL * CH, TAIL)]], tra, sT).wait()
        pltpu.async_copy(b_h.at[iss.at[pl.ds(NFULL * CH, TAIL)]], tra, sT, add=True)
        pltpu.make_async_copy(b_h.at[iss.at[pl.ds(NFULL * CH, TAIL)]], tra, sT).wait()
        pltpu.sync_copy(tra, out_h.at[pl.ds(base, TAIL)])

    return k(A, B, src, dst)


def _edge_mlp_packed(m1p, W2big, b2big):
    PW = W2big.shape[1]
    BE4 = 2000

    def body(x_ref, w_ref, bb_ref, o_ref):
        m1 = jnp.maximum(x_ref[...], 0.0)
        o_ref[...] = jnp.maximum(m1 @ w_ref[...] + bb_ref[...], 0.0)

    return pl.pallas_call(
        body,
        grid=(E4 // BE4,),
        in_specs=[
            pl.BlockSpec((BE4, 128), lambda i: (i, 0)),
            pl.BlockSpec((128, PW), lambda i: (0, 0)),
            pl.BlockSpec((1, PW), lambda i: (0, 0)),
        ],
        out_specs=pl.BlockSpec((BE4, PW), lambda i: (i, 0)),
        out_shape=jax.ShapeDtypeStruct((E4, PW), jnp.float32),
    )(m1p, W2big, b2big)


def _sc_segmax(msg_p, dst, feature_split=False):
    PW = msg_p.shape[1]
    Fh = 16
    npass = 2
    RANGE = N // npass
    CHS = 400
    CHS4 = CHS // 4
    EPWk = (E // NS) if feature_split else EPW
    NCHS = EPWk // CHS
    MSUB = RANGE // 16
    MLAST = RANGE - 15 * MSUB
    WS = 4 if PW == 64 else 2
    mesh = plsc.VectorSubcoreMesh(core_axis_name="c", subcore_axis_name="s")

    @functools.partial(
        pl.kernel,
        out_type=pltpu.HBM((NC, N, Fh), jnp.float32),
        mesh=mesh,
        scratch_types=[
            pltpu.VMEM((CHS,), jnp.int32),
            pltpu.VMEM((CHS,), jnp.int32),
            pltpu.VMEM((CHS4, PW), jnp.float32),
            pltpu.VMEM((CHS4, PW), jnp.float32),
            pltpu.VMEM((RANGE + 8, Fh), jnp.float32),
            pltpu.VMEM((MLAST, Fh), jnp.float32),
            pltpu.VMEM((MLAST, Fh), jnp.float32),
            pltpu.VMEM_SHARED((WS, RANGE, Fh), jnp.float32),
            pltpu.SemaphoreType.DMA,
            pltpu.SemaphoreType.DMA,
            pltpu.SemaphoreType.DMA,
            pltpu.SemaphoreType.DMA,
        ],
        compiler_params=pltpu.CompilerParams(use_tc_tiling_on_sc=False),
    )
    def k(msg_h, dst_h, parts_h, dv0, dv1, mv0, mv1, acc, sbuf, mbuf, shm,
          sD0, sD1, sM0, sM1):
        cid = lax.axis_index("c")
        sid = lax.axis_index("s")
        if feature_split:
            ebase = sid * EPWk
            colofs = cid * Fh
        else:
            ebase = (sid * NC + cid) * EPWk
            colofs = 0
        rbase = (ebase // 4)
        dvs = (dv0, dv1)
        mvs = (mv0, mv1)
        sDs = (sD0, sD1)
        sMs = (sM0, sM1)

        def fire(ci, p):
            pltpu.async_copy(dst_h.at[pl.ds(ebase + ci * CHS, CHS)], dvs[p], sDs[p])
            pltpu.async_copy(msg_h.at[pl.ds(rbase + ci * CHS4, CHS4)], mvs[p], sMs[p])

        def wait_in(ci, p):
            pltpu.make_async_copy(dst_h.at[pl.ds(ebase + ci * CHS, CHS)], dvs[p], sDs[p]).wait()
            pltpu.make_async_copy(msg_h.at[pl.ds(rbase + ci * CHS4, CHS4)], mvs[p], sMs[p]).wait()

        def do_pass(pp, _):
            nbase = pp * RANGE

            def zero(i, _):
                acc[i, pl.ds(0, 16)] = jnp.zeros((16,), jnp.float32)
                return _

            lax.fori_loop(0, RANGE + 8, zero, None)

            def rmw_chunk(ci, p):
                dv = dvs[p]
                mv = mvs[p]

                def group(gi, _):
                    dvec = dv[pl.ds(gi * 16, 16)] - nbase
                    inr = jnp.logical_and(dvec >= 0, dvec < RANGE)
                    lsel = jnp.where(inr, dvec, RANGE)
                    for lane in range(16):
                        local = lsel[lane]
                        r = gi * 4 + (lane // 4)
                        c = (lane % 4) * (PW // 4) + colofs
                        old = acc[local, pl.ds(0, 16)]
                        row = mv[r, pl.ds(c, 16)]
                        acc[local, pl.ds(0, 16)] = jnp.maximum(old, row)

                    return _

                lax.fori_loop(0, CHS // 16, group, None)

            fire(0, 0)

            def cpair(it, _):
                ci = it * 2
                fire(ci + 1, 1)
                wait_in(ci, 0)
                rmw_chunk(ci, 0)

                @pl.when(ci + 2 < NCHS)
                def _():
                    fire(ci + 2, 0)

                wait_in(ci + 1, 1)
                rmw_chunk(ci + 1, 1)
                return _

            lax.fori_loop(0, NCHS // 2, cpair, None)
            if NCHS % 2 == 1:
                wait_in(NCHS - 1, 0)
                rmw_chunk(NCHS - 1, 0)

            roff = sid * MSUB

            def merge_rows(nrows, roff, zero_first):
                if zero_first:
                    def mz(i, _):
                        mbuf[i, pl.ds(0, 16)] = jnp.zeros((16,), jnp.float32)
                        return _

                    lax.fori_loop(0, nrows, mz, None)

                def slot(t, _):
                    pltpu.sync_copy(shm.at[t, pl.ds(roff, nrows)],
                                    sbuf.at[pl.ds(0, nrows)])

                    def row(i, _):
                        mbuf[i, pl.ds(0, 16)] = jnp.maximum(
                            mbuf[i, pl.ds(0, 16)],
                            sbuf[i, pl.ds(0, 16)])
                        return _

                    lax.fori_loop(0, nrows, row, None)
                    return _

                lax.fori_loop(0, WS, slot, None)

            for w in range(NS // WS):
                @pl.when(sid // WS == w)
                def _(w=w):
                    pltpu.sync_copy(acc.at[pl.ds(0, RANGE)], shm.at[sid % WS])

                plsc.subcore_barrier()

                @pl.when(sid < 15)
                def _(w=w):
                    merge_rows(MSUB, roff, w == 0)

                @pl.when(sid == 15)
                def _(w=w):
                    merge_rows(MLAST, roff, w == 0)

                plsc.subcore_barrier()

            @pl.when(sid < 15)
            def _():
                pltpu.sync_copy(mbuf.at[pl.ds(0, MSUB)],
                                parts_h.at[cid, pl.ds(nbase + roff, MSUB)])

            @pl.when(sid == 15)
            def _():
                pltpu.sync_copy(mbuf.at[pl.ds(0, MLAST)],
                                parts_h.at[cid, pl.ds(nbase + roff, MLAST)])
            return _

        lax.fori_loop(0, npass, do_pass, None)

    return k(msg_p, dst)


def _head(parts3, batch2d, lin1_W, lin1_b, lin2_W, lin2_b):
    P, n, Fhp = parts3.shape
    BN = 1000
    steps = n // BN
    Fh = P * Fhp
    F1 = lin1_W.shape[1]
    F2 = lin2_W.shape[1]

    def body(p_ref, b_ref, w1_ref, b1_ref, w2_ref, b2_ref, o_ref, sums, cnts):
        i = pl.program_id(0)

        @pl.when(i == 0)
        def _init():
            sums[...] = jnp.zeros_like(sums)
            cnts[...] = jnp.zeros_like(cnts)

        h = jnp.concatenate([p_ref[0], p_ref[1]], axis=-1)
        bb = b_ref[...]
        ids = lax.broadcasted_iota(jnp.int32, (1, NG), 1).astype(jnp.float32)
        oh = (bb == ids).astype(jnp.float32)
        dn = (((0,), (0,)), ((), ()))
        sums[...] += lax.dot_general(oh, h, dn)
        cnts[...] += lax.dot_general(oh, jnp.ones((BN, 1), jnp.float32), dn)

        @pl.when(i == steps - 1)
        def _fin():
            g = sums[...] / jnp.maximum(cnts[...], 1.0)
            z = jnp.maximum(g @ w1_ref[...] + b1_ref[...], 0.0)
            z = jnp.maximum(z @ w2_ref[...] + b2_ref[...], 0.0)
            m = jnp.max(z, axis=-1, keepdims=True)
            lse = m + jnp.log(jnp.sum(jnp.exp(z - m), axis=-1, keepdims=True))
            o_ref[...] = z - lse

    return pl.pallas_call(
        body,
        grid=(steps,),
        in_specs=[
            pl.BlockSpec((P, BN, Fhp), lambda i: (0, i, 0)),
            pl.BlockSpec((BN, 1), lambda i: (i, 0)),
            pl.BlockSpec((Fh, F1), lambda i: (0, 0)),
            pl.BlockSpec((1, F1), lambda i: (0, 0)),
            pl.BlockSpec((F1, F2), lambda i: (0, 0)),
            pl.BlockSpec((1, F2), lambda i: (0, 0)),
        ],
        out_specs=pl.BlockSpec((NG, F2), lambda i: (0, 0)),
        out_shape=jax.ShapeDtypeStruct((NG, F2), jnp.float32),
        scratch_shapes=[
            pltpu.VMEM((NG, Fh), jnp.float32),
            pltpu.VMEM((NG, 1), jnp.float32),
        ],
    )(parts3, batch2d, lin1_W, lin1_b.reshape(1, F1), lin2_W, lin2_b.reshape(1, F2))


def _big_w2(W2, b2):
    H, F2 = W2.shape
    if F2 < 32:
        W2 = jnp.pad(W2, ((0, 0), (0, 32 - F2)))
        b2 = jnp.pad(b2, (0, 32 - F2))
        F2 = 32
    Wb = jnp.zeros((128, 4 * F2), jnp.float32)
    for j in range(4):
        Wb = Wb.at[j * H:(j + 1) * H, j * F2:(j + 1) * F2].set(W2)
    bb = jnp.tile(b2, (4,)).reshape(1, 4 * F2)
    return Wb, bb


def _layer(parts, src, dst, W1, b1, W2, b2, feature_split):
    A, B = _node_transform(parts, W1, b1)
    m1raw = _sc_gather_combine(A, B, src, dst)
    m1p = m1raw.reshape(E4, 128)
    W2big, b2big = _big_w2(W2, b2)
    msg_p = _edge_mlp_packed(m1p, W2big, b2big)
    return _sc_segmax(msg_p, dst, feature_split)


@jax.jit
def kernel(x, edge_index, batch,
           eg1_W1, eg1_b1, eg1_W2, eg1_b2,
           eg2_W1, eg2_b1, eg2_W2, eg2_b2,
           eg3_W1, eg3_b1, eg3_W2, eg3_b2,
           lin1_W, lin1_b, lin2_W, lin2_b):
    src = edge_index[0]
    dst = edge_index[1]
    parts = x.reshape(1, N, x.shape[1])
    parts = _layer(parts, src, dst, eg1_W1, eg1_b1, eg1_W2, eg1_b2, False)
    parts = _layer(parts, src, dst, eg2_W1, eg2_b1, eg2_W2, eg2_b2, False)
    parts = _layer(parts, src, dst, eg3_W1, eg3_b1, eg3_W2, eg3_b2, True)
    batch2d = batch.astype(jnp.float32).reshape(N, 1)
    return _head(parts, batch2d, lin1_W, lin1_b, lin2_W, lin2_b)
